# Initial kernel scaffold; baseline (speedup 1.0000x reference)
import jax
import jax.numpy as jnp
from jax.experimental import pallas as pl


def kernel(token, row_embed, col_embed):
    raise NotImplementedError("write your pallas kernel here")


# speedup vs baseline: 1.9505x; 1.9505x over previous
import functools

import jax
import jax.numpy as jnp
from jax import lax
from jax.experimental import pallas as pl
from jax.experimental.pallas import tpu as pltpu
from jax.experimental.pallas import tpu_sc as plsc

H = 512
HALF = 384

_info = plsc.get_sparse_core_info()
_NC, _NS, _L = _info.num_cores, _info.num_subcores, _info.num_lanes
_NW = _NC * _NS


def _make_gather(n_rows: int):
  per_w = n_rows // _NW
  R = 64
  assert per_w % R == 0
  G = per_w // R
  assert G % 2 == 0
  mesh = plsc.VectorSubcoreMesh(core_axis_name="c", subcore_axis_name="s")

  @functools.partial(
      pl.kernel,
      mesh=mesh,
      out_type=jax.ShapeDtypeStruct((n_rows, HALF), jnp.float32),
      scratch_types=[
          pltpu.VMEM((2, R), jnp.int32),
          pltpu.VMEM((2, R, HALF), jnp.float32),
          pltpu.SemaphoreType.DMA,
          pltpu.SemaphoreType.DMA,
          pltpu.SemaphoreType.DMA,
          pltpu.SemaphoreType.DMA,
      ],
  )
  def k(table_hbm, idx_hbm, out_hbm, idx_v, rows_v, g0, g1, w0, w1):
    wid = lax.axis_index("s") * _NC + lax.axis_index("c")
    base = wid * per_w
    offs = (lax.iota(jnp.int32, _L) & 1) * H
    gsem = (g0, g1)
    wsem = (w0, w1)

    def load_idx(g, b):
      pltpu.sync_copy(idx_hbm.at[pl.ds(base + g * R, R)], idx_v.at[b])
      for i in range(R // _L):
        sl = pl.ds(i * _L, _L)
        idx_v[b, sl] = idx_v[b, sl] + offs

    def start_gather(b):
      return pltpu.async_copy(table_hbm.at[idx_v.at[b]], rows_v.at[b], gsem[b])

    def wait_gather(b):
      pltpu.make_async_copy(
          table_hbm.at[idx_v.at[b]], rows_v.at[b], gsem[b]).wait()

    def start_write(g, b):
      return pltpu.async_copy(
          rows_v.at[b], out_hbm.at[pl.ds(base + g * R, R)], wsem[b])

    load_idx(0, 0)
    start_gather(0)
    load_idx(1, 1)
    start_gather(1)

    @pl.loop(0, G - 2, step=2)
    def _(g):
      for b in range(2):
        wait_gather(b)
        wcopy = start_write(g + b, b)
        load_idx(g + b + 2, b)
        wcopy.wait()
        start_gather(b)

    wait_gather(0)
    wl0 = start_write(G - 2, 0)
    wait_gather(1)
    wl1 = start_write(G - 1, 1)
    wl0.wait()
    wl1.wait()

  return k


_gather = _make_gather(2 * 1024 * 50)


def kernel(token, row_embed, col_embed):
  B, L, _ = token.shape
  table = jnp.concatenate([row_embed, col_embed], axis=0)
  idx = token.astype(jnp.int32).reshape(-1)
  out = _gather(table, idx)
  return out.reshape(B, L, 2 * HALF)
